# Initial kernel scaffold; baseline (speedup 1.0000x reference)
import jax
import jax.numpy as jnp
from jax.experimental import pallas as pl


def kernel(hidden_states, router_indices, routing_weights, gate_up_proj, gate_up_proj_bias, down_proj, down_proj_bias):
    raise NotImplementedError("write your pallas kernel here")


# speedup vs baseline: 3.6024x; 3.6024x over previous
import functools

import jax
import jax.numpy as jnp
from jax.experimental import pallas as pl
from jax.experimental.pallas import tpu as pltpu

HIDDEN = 1024
INTER = 1024
ALPHA = 1.702
LIMIT = 7.0


def _experts_kernel(hs_ref, wgu_ref, wd_ref, bgu_ref, bd_ref, rw_ref, out_ref,
                    *, bt: int, n_experts: int):
    e = pl.program_id(0)
    t = pl.program_id(1)

    x = hs_ref[...]
    gu = jnp.dot(x, wgu_ref[0], preferred_element_type=jnp.float32)
    gu = gu + bgu_ref[0]
    gate = gu[:, :INTER]
    up = gu[:, INTER:]
    gate = jnp.minimum(gate, LIMIT)
    up = jnp.clip(up, -LIMIT, LIMIT)
    glu = gate * jax.nn.sigmoid(gate * ALPHA)
    act = (up + 1.0) * glu
    nxt = jnp.dot(act, wd_ref[0], preferred_element_type=jnp.float32)
    nxt = (nxt + bd_ref[0]) * rw_ref[0]

    sl = pl.ds(t * bt, bt)

    @pl.when(e == 0)
    def _init():
        out_ref[sl, :] = nxt

    @pl.when(e > 0)
    def _acc():
        out_ref[sl, :] = out_ref[sl, :] + nxt


def kernel(hidden_states, router_indices, routing_weights, gate_up_proj,
           gate_up_proj_bias, down_proj, down_proj_bias):
    del router_indices
    T, H = hidden_states.shape
    E = routing_weights.shape[1]
    BT = 512
    NT = T // BT

    wgu = jnp.concatenate(
        [gate_up_proj[:, :, 0::2], gate_up_proj[:, :, 1::2]], axis=-1)
    bgu = jnp.concatenate(
        [gate_up_proj_bias[:, 0::2], gate_up_proj_bias[:, 1::2]],
        axis=-1).reshape(E, 1, 2 * INTER)
    bd = down_proj_bias.reshape(E, 1, HIDDEN)
    rw = routing_weights.T.reshape(E, T, 1)

    grid = (E, NT)
    out = pl.pallas_call(
        functools.partial(_experts_kernel, bt=BT, n_experts=E),
        grid=grid,
        in_specs=[
            pl.BlockSpec((BT, H), lambda e, t: (t, 0)),
            pl.BlockSpec((1, H, 2 * INTER), lambda e, t: (e, 0, 0)),
            pl.BlockSpec((1, INTER, H), lambda e, t: (e, 0, 0)),
            pl.BlockSpec((1, 1, 2 * INTER), lambda e, t: (e, 0, 0)),
            pl.BlockSpec((1, 1, H), lambda e, t: (e, 0, 0)),
            pl.BlockSpec((1, BT, 1), lambda e, t: (e, t, 0)),
        ],
        out_specs=pl.BlockSpec((T, H), lambda e, t: (0, 0)),
        out_shape=jax.ShapeDtypeStruct((T, H), jnp.float32),
        compiler_params=pltpu.CompilerParams(
            dimension_semantics=("arbitrary", "arbitrary"),
        ),
    )(hidden_states, wgu, down_proj, bgu, bd, rw)
    return out


# speedup vs baseline: 6.1343x; 1.7028x over previous
import functools

import jax
import jax.numpy as jnp
from jax.experimental import pallas as pl
from jax.experimental.pallas import tpu as pltpu

HIDDEN = 1024
INTER = 1024
ALPHA = 1.702
LIMIT = 7.0


def _experts_kernel(hs_ref, wgu_ref, wd_ref, bgu_ref, bd_ref, rw_ref, out_ref,
                    *, bt: int, n_experts: int):
    e = pl.program_id(0)
    t = pl.program_id(1)

    x = hs_ref[...]
    gu = jnp.dot(x, wgu_ref[0], preferred_element_type=jnp.float32)
    gu = gu + bgu_ref[0]
    gate = gu[:, :INTER]
    up = gu[:, INTER:]
    gate = jnp.minimum(gate, LIMIT)
    up = jnp.clip(up, -LIMIT, LIMIT)
    glu = gate * jax.nn.sigmoid(gate * ALPHA)
    act = ((up + 1.0) * glu).astype(jnp.bfloat16)
    nxt = jnp.dot(act, wd_ref[0], preferred_element_type=jnp.float32)
    nxt = (nxt + bd_ref[0]) * rw_ref[0]

    sl = pl.ds(t * bt, bt)

    @pl.when(e == 0)
    def _init():
        out_ref[sl, :] = nxt

    @pl.when(e > 0)
    def _acc():
        out_ref[sl, :] = out_ref[sl, :] + nxt


def kernel(hidden_states, router_indices, routing_weights, gate_up_proj,
           gate_up_proj_bias, down_proj, down_proj_bias):
    del router_indices
    T, H = hidden_states.shape
    E = routing_weights.shape[1]
    BT = 512
    NT = T // BT

    wgu = jnp.concatenate(
        [gate_up_proj[:, :, 0::2], gate_up_proj[:, :, 1::2]],
        axis=-1).astype(jnp.bfloat16)
    wd = down_proj.astype(jnp.bfloat16)
    hs = hidden_states.astype(jnp.bfloat16)
    bgu = jnp.concatenate(
        [gate_up_proj_bias[:, 0::2], gate_up_proj_bias[:, 1::2]],
        axis=-1).reshape(E, 1, 2 * INTER)
    bd = down_proj_bias.reshape(E, 1, HIDDEN)
    rw = routing_weights.T.reshape(E, T, 1)

    grid = (E, NT)
    out = pl.pallas_call(
        functools.partial(_experts_kernel, bt=BT, n_experts=E),
        grid=grid,
        in_specs=[
            pl.BlockSpec((BT, H), lambda e, t: (t, 0)),
            pl.BlockSpec((1, H, 2 * INTER), lambda e, t: (e, 0, 0)),
            pl.BlockSpec((1, INTER, H), lambda e, t: (e, 0, 0)),
            pl.BlockSpec((1, 1, 2 * INTER), lambda e, t: (e, 0, 0)),
            pl.BlockSpec((1, 1, H), lambda e, t: (e, 0, 0)),
            pl.BlockSpec((1, BT, 1), lambda e, t: (e, t, 0)),
        ],
        out_specs=pl.BlockSpec((T, H), lambda e, t: (0, 0)),
        out_shape=jax.ShapeDtypeStruct((T, H), jnp.float32),
        compiler_params=pltpu.CompilerParams(
            dimension_semantics=("arbitrary", "arbitrary"),
        ),
    )(hs, wgu, wd, bgu, bd, rw)
    return out


# speedup vs baseline: 19.1404x; 3.1202x over previous
import functools

import jax
import jax.numpy as jnp
from jax.experimental import pallas as pl
from jax.experimental.pallas import tpu as pltpu

HIDDEN = 1024
INTER = 1024
ALPHA = 1.702
LIMIT = 7.0
NC = 2
CW = 2 * INTER // NC
IW = INTER // NC


def _experts_kernel(hs_ref, wgu_ref, wd_ref, bgu_ref, bd_ref, rw_ref, out_ref,
                    wgu_s, wd_s, *, bt: int):
    e = pl.program_id(0)
    c = pl.program_id(1)
    t = pl.program_id(2)

    @pl.when(t == 0)
    def _cast_weights():
        wgu_s[...] = wgu_ref[0].astype(jnp.bfloat16)
        wd_s[...] = jnp.repeat(wd_ref[0].astype(jnp.bfloat16), 2, axis=0)

    sl = pl.ds(t * bt, bt)
    x = hs_ref[sl, :].astype(jnp.bfloat16)
    gu = jnp.dot(x, wgu_s[...], preferred_element_type=jnp.float32)
    gu = gu + bgu_ref[0]
    gu_next = pltpu.roll(gu, CW - 1, 1)
    gate = jnp.minimum(gu, LIMIT)
    up = jnp.clip(gu_next, -LIMIT, LIMIT)
    glu = gate * jax.nn.sigmoid(gate * ALPHA)
    act_w = (up + 1.0) * glu
    lane = jax.lax.broadcasted_iota(jnp.int32, act_w.shape, 1)
    act_w = jnp.where(lane % 2 == 0, act_w, 0.0).astype(jnp.bfloat16)
    nxt = jnp.dot(act_w, wd_s[...], preferred_element_type=jnp.float32)
    nxt = nxt * rw_ref[0]

    first = jnp.logical_and(e == 0, c == 0)

    @pl.when(first)
    def _init():
        out_ref[sl, :] = nxt + bd_ref[0] * rw_ref[0]

    @pl.when(jnp.logical_not(first))
    def _acc():
        acc = out_ref[sl, :] + nxt

        @pl.when(c == 0)
        def _bias():
            out_ref[sl, :] = acc + bd_ref[0] * rw_ref[0]

        @pl.when(c != 0)
        def _nobias():
            out_ref[sl, :] = acc


def kernel(hidden_states, router_indices, routing_weights, gate_up_proj,
           gate_up_proj_bias, down_proj, down_proj_bias):
    del router_indices
    T, H = hidden_states.shape
    E = routing_weights.shape[1]
    BT = 512
    NT = T // BT

    bgu = gate_up_proj_bias.reshape(E, 1, 2 * INTER)
    bd = down_proj_bias.reshape(E, 1, HIDDEN)
    rw = routing_weights.T.reshape(E, T, 1)

    grid = (E, NC, NT)
    out = pl.pallas_call(
        functools.partial(_experts_kernel, bt=BT),
        grid=grid,
        in_specs=[
            pl.BlockSpec((T, H), lambda e, c, t: (0, 0)),
            pl.BlockSpec((1, H, CW), lambda e, c, t: (e, 0, c)),
            pl.BlockSpec((1, IW, H), lambda e, c, t: (e, c, 0)),
            pl.BlockSpec((1, 1, CW), lambda e, c, t: (e, 0, c)),
            pl.BlockSpec((1, 1, H), lambda e, c, t: (e, 0, 0)),
            pl.BlockSpec((1, BT, 1), lambda e, c, t: (e, t, 0)),
        ],
        out_specs=pl.BlockSpec((T, H), lambda e, c, t: (0, 0)),
        out_shape=jax.ShapeDtypeStruct((T, H), jnp.float32),
        scratch_shapes=[
            pltpu.VMEM((HIDDEN, CW), jnp.bfloat16),
            pltpu.VMEM((2 * IW, HIDDEN), jnp.bfloat16),
        ],
        compiler_params=pltpu.CompilerParams(
            dimension_semantics=("arbitrary", "arbitrary", "arbitrary"),
            vmem_limit_bytes=64 * 1024 * 1024,
        ),
    )(hidden_states, gate_up_proj, down_proj, bgu, bd, rw)
    return out


# speedup vs baseline: 23.7554x; 1.2411x over previous
import functools

import jax
import jax.numpy as jnp
from jax.experimental import pallas as pl
from jax.experimental.pallas import tpu as pltpu

HIDDEN = 1024
INTER = 1024
ALPHA = 1.702
LIMIT = 7.0
NC = 2
CW = 2 * INTER // NC
IW = INTER // NC


def _experts_kernel(hs_ref, wgu_ref, wd_ref, bgu_ref, bd_ref, rw_ref, out_ref,
                    hs_bf, wgu_s, wd_s, p_s, *, bt: int, nt: int):
    e = pl.program_id(0)
    c = pl.program_id(1)
    t = pl.program_id(2)
    first_ec = jnp.logical_and(e == 0, c == 0)

    @pl.when(jnp.logical_and(first_ec, t == 0))
    def _build_perm():
        k = jax.lax.broadcasted_iota(jnp.int32, (CW, CW), 0)
        j = jax.lax.broadcasted_iota(jnp.int32, (CW, CW), 1)
        src = jnp.where(j < IW, 2 * j, 2 * (j - IW) + 1)
        p_s[...] = (k == src).astype(jnp.bfloat16)

    @pl.when(first_ec)
    def _cast_tokens():
        hs_bf[pl.ds(t * bt, bt), :] = hs_ref[...].astype(jnp.bfloat16)

    @pl.when(t == 0)
    def _prep_weights():
        wgu_s[...] = jnp.dot(wgu_ref[0].astype(jnp.bfloat16), p_s[...],
                             preferred_element_type=jnp.float32
                             ).astype(jnp.bfloat16)
        wd_s[...] = wd_ref[0].astype(jnp.bfloat16)

    sl = pl.ds(t * bt, bt)
    x = hs_bf[sl, :]
    gu = jnp.dot(x, wgu_s[...], preferred_element_type=jnp.float32)
    gu = gu + bgu_ref[0, 0]
    gate = gu[:, :IW]
    up = gu[:, IW:]
    gate = jnp.minimum(gate, LIMIT)
    up = jnp.clip(up, -LIMIT, LIMIT)
    glu = gate * jax.nn.sigmoid(gate * ALPHA)
    act = ((up + 1.0) * glu).astype(jnp.bfloat16)
    nxt = jnp.dot(act, wd_s[...], preferred_element_type=jnp.float32)
    nxt = nxt * rw_ref[0]

    @pl.when(first_ec)
    def _init():
        out_ref[sl, :] = nxt + bd_ref[0] * rw_ref[0]

    @pl.when(jnp.logical_not(first_ec))
    def _acc():
        acc = out_ref[sl, :] + nxt

        @pl.when(c == 0)
        def _bias():
            out_ref[sl, :] = acc + bd_ref[0] * rw_ref[0]

        @pl.when(c != 0)
        def _nobias():
            out_ref[sl, :] = acc


def kernel(hidden_states, router_indices, routing_weights, gate_up_proj,
           gate_up_proj_bias, down_proj, down_proj_bias):
    del router_indices
    T, H = hidden_states.shape
    E = routing_weights.shape[1]
    BT = 512
    NT = T // BT

    b3 = gate_up_proj_bias.reshape(E, NC, 1, CW)
    bgu = jnp.concatenate([b3[..., 0::2], b3[..., 1::2]], axis=-1)
    bd = down_proj_bias.reshape(E, 1, HIDDEN)
    rw = routing_weights.T.reshape(E, T, 1)

    def hs_idx(e, c, t):
        first_ec = jnp.logical_and(e == 0, c == 0)
        return (jnp.where(first_ec, t, NT - 1), 0)

    grid = (E, NC, NT)
    out = pl.pallas_call(
        functools.partial(_experts_kernel, bt=BT, nt=NT),
        grid=grid,
        in_specs=[
            pl.BlockSpec((BT, H), hs_idx),
            pl.BlockSpec((1, H, CW), lambda e, c, t: (e, 0, c)),
            pl.BlockSpec((1, IW, H), lambda e, c, t: (e, c, 0)),
            pl.BlockSpec((1, 1, 1, CW), lambda e, c, t: (e, c, 0, 0)),
            pl.BlockSpec((1, 1, H), lambda e, c, t: (e, 0, 0)),
            pl.BlockSpec((1, BT, 1), lambda e, c, t: (e, t, 0)),
        ],
        out_specs=pl.BlockSpec((T, H), lambda e, c, t: (0, 0)),
        out_shape=jax.ShapeDtypeStruct((T, H), jnp.float32),
        scratch_shapes=[
            pltpu.VMEM((T, HIDDEN), jnp.bfloat16),
            pltpu.VMEM((HIDDEN, CW), jnp.bfloat16),
            pltpu.VMEM((IW, HIDDEN), jnp.bfloat16),
            pltpu.VMEM((CW, CW), jnp.bfloat16),
        ],
        compiler_params=pltpu.CompilerParams(
            dimension_semantics=("arbitrary", "arbitrary", "arbitrary"),
            vmem_limit_bytes=64 * 1024 * 1024,
        ),
    )(hidden_states, gate_up_proj, down_proj, bgu, bd, rw)
    return out


# speedup vs baseline: 24.0217x; 1.0112x over previous
import functools

import jax
import jax.numpy as jnp
from jax.experimental import pallas as pl
from jax.experimental.pallas import tpu as pltpu

HIDDEN = 1024
INTER = 1024
ALPHA = 1.702
LIMIT = 7.0
NC = 2
CW = 2 * INTER // NC
IW = INTER // NC


def _experts_kernel(hs_ref, wgu_ref, wd_ref, bgu_ref, bd_ref, rw_ref,
                    rwf_ref, out_ref, hs_bf, wgu_s, wd_s, p_s,
                    *, bt: int, nt: int):
    e = pl.program_id(0)
    c = pl.program_id(1)
    t = pl.program_id(2)
    first_ec = jnp.logical_and(e == 0, c == 0)

    @pl.when(jnp.logical_and(first_ec, t == 0))
    def _build_perm():
        k = jax.lax.broadcasted_iota(jnp.int32, (CW, CW), 0)
        j = jax.lax.broadcasted_iota(jnp.int32, (CW, CW), 1)
        src = jnp.where(j < IW, 2 * j, 2 * (j - IW) + 1)
        p_s[...] = (k == src).astype(jnp.bfloat16)

    @pl.when(first_ec)
    def _cast_tokens():
        hs_bf[pl.ds(t * bt, bt), :] = hs_ref[...].astype(jnp.bfloat16)

    @pl.when(t == 0)
    def _prep_weights():
        wgu_s[...] = jnp.dot(wgu_ref[0].astype(jnp.bfloat16), p_s[...],
                             preferred_element_type=jnp.float32
                             ).astype(jnp.bfloat16)
        wd_s[...] = wd_ref[0].astype(jnp.bfloat16)

    sl = pl.ds(t * bt, bt)
    x = hs_bf[sl, :]
    gu = jnp.dot(x, wgu_s[...], preferred_element_type=jnp.float32)
    gu = gu + bgu_ref[0, 0]
    gate = gu[:, :IW]
    up = gu[:, IW:]
    gate = jnp.minimum(gate, LIMIT)
    up = jnp.clip(up, -LIMIT, LIMIT)
    glu = gate * jax.nn.sigmoid(gate * ALPHA)
    act = ((up + 1.0) * glu * rw_ref[0]).astype(jnp.bfloat16)
    nxt = jnp.dot(act, wd_s[...], preferred_element_type=jnp.float32)

    @pl.when(first_ec)
    def _init():
        bias = jnp.dot(rwf_ref[...], bd_ref[...],
                       preferred_element_type=jnp.float32)
        out_ref[sl, :] = nxt + bias

    @pl.when(jnp.logical_not(first_ec))
    def _acc():
        out_ref[sl, :] = out_ref[sl, :] + nxt


def kernel(hidden_states, router_indices, routing_weights, gate_up_proj,
           gate_up_proj_bias, down_proj, down_proj_bias):
    del router_indices
    T, H = hidden_states.shape
    E = routing_weights.shape[1]
    BT = 512
    NT = T // BT

    b3 = gate_up_proj_bias.reshape(E, NC, 1, CW)
    bgu = jnp.concatenate([b3[..., 0::2], b3[..., 1::2]], axis=-1)
    rw = routing_weights.T.reshape(E, T, 1)

    def hs_idx(e, c, t):
        first_ec = jnp.logical_and(e == 0, c == 0)
        return (jnp.where(first_ec, t, NT - 1), 0)

    grid = (E, NC, NT)
    out = pl.pallas_call(
        functools.partial(_experts_kernel, bt=BT, nt=NT),
        grid=grid,
        in_specs=[
            pl.BlockSpec((BT, H), hs_idx),
            pl.BlockSpec((1, H, CW), lambda e, c, t: (e, 0, c)),
            pl.BlockSpec((1, IW, H), lambda e, c, t: (e, c, 0)),
            pl.BlockSpec((1, 1, 1, CW), lambda e, c, t: (e, c, 0, 0)),
            pl.BlockSpec((E, H), lambda e, c, t: (0, 0)),
            pl.BlockSpec((1, BT, 1), lambda e, c, t: (e, t, 0)),
            pl.BlockSpec((BT, E), lambda e, c, t: (t, 0)),
        ],
        out_specs=pl.BlockSpec((T, H), lambda e, c, t: (0, 0)),
        out_shape=jax.ShapeDtypeStruct((T, H), jnp.float32),
        scratch_shapes=[
            pltpu.VMEM((T, HIDDEN), jnp.bfloat16),
            pltpu.VMEM((HIDDEN, CW), jnp.bfloat16),
            pltpu.VMEM((IW, HIDDEN), jnp.bfloat16),
            pltpu.VMEM((CW, CW), jnp.bfloat16),
        ],
        compiler_params=pltpu.CompilerParams(
            dimension_semantics=("arbitrary", "arbitrary", "arbitrary"),
            vmem_limit_bytes=64 * 1024 * 1024,
        ),
    )(hidden_states, gate_up_proj, down_proj, bgu, down_proj_bias, rw,
      routing_weights)
    return out


# speedup vs baseline: 25.5354x; 1.0630x over previous
import functools

import jax
import jax.numpy as jnp
from jax.experimental import pallas as pl
from jax.experimental.pallas import tpu as pltpu

HIDDEN = 1024
INTER = 1024
ALPHA = 1.702
LIMIT = 7.0
NC = 2
CW = 2 * INTER // NC
IW = INTER // NC


def _experts_kernel(hs_ref, wgu_ref, wd_ref, bgu_ref, bd_ref, rw_ref,
                    rwf_ref, out_ref, hs_bf, wgu_s, wd_s, p_s,
                    *, bt: int, nt: int):
    e = pl.program_id(0)
    c = pl.program_id(1)
    t = pl.program_id(2)
    first_ec = jnp.logical_and(e == 0, c == 0)

    @pl.when(jnp.logical_and(first_ec, t == 0))
    def _build_perm():
        k = jax.lax.broadcasted_iota(jnp.int32, (CW, CW), 0)
        j = jax.lax.broadcasted_iota(jnp.int32, (CW, CW), 1)
        src = jnp.where(j < IW, 2 * j, 2 * (j - IW) + 1)
        p_s[...] = (k == src).astype(jnp.bfloat16)

    @pl.when(first_ec)
    def _cast_tokens():
        hs_bf[pl.ds(t * bt, bt), :] = hs_ref[...].astype(jnp.bfloat16)

    @pl.when(t == 0)
    def _prep_weights():
        wgu_s[...] = jnp.dot(wgu_ref[0].astype(jnp.bfloat16), p_s[...],
                             preferred_element_type=jnp.float32
                             ).astype(jnp.bfloat16)
        wd_s[...] = wd_ref[0].astype(jnp.bfloat16)

    sl = pl.ds(t * bt, bt)
    x = hs_bf[sl, :]
    gu = jnp.dot(x, wgu_s[...], preferred_element_type=jnp.float32)
    gu = gu + bgu_ref[0, 0]
    gate = gu[:, :IW]
    up = gu[:, IW:]
    gate = jnp.minimum(gate, LIMIT)
    up = jnp.clip(up, -LIMIT, LIMIT)
    glu = gate * jax.nn.sigmoid(gate * ALPHA)
    act = ((up + 1.0) * glu * rw_ref[0]).astype(jnp.bfloat16)
    nxt = jnp.dot(act, wd_s[...], preferred_element_type=jnp.float32)

    @pl.when(first_ec)
    def _init():
        bias = jnp.dot(rwf_ref[...], bd_ref[...],
                       preferred_element_type=jnp.float32)
        out_ref[sl, :] = nxt + bias

    @pl.when(jnp.logical_not(first_ec))
    def _acc():
        out_ref[sl, :] = out_ref[sl, :] + nxt


def kernel(hidden_states, router_indices, routing_weights, gate_up_proj,
           gate_up_proj_bias, down_proj, down_proj_bias):
    del router_indices
    T, H = hidden_states.shape
    E = routing_weights.shape[1]
    BT = 1024
    NT = T // BT

    b3 = gate_up_proj_bias.reshape(E, NC, 1, CW)
    bgu = jnp.concatenate([b3[..., 0::2], b3[..., 1::2]], axis=-1)
    rw = routing_weights.T.reshape(E, T, 1)

    def hs_idx(e, c, t):
        first_ec = jnp.logical_and(e == 0, c == 0)
        return (jnp.where(first_ec, t, NT - 1), 0)

    grid = (E, NC, NT)
    out = pl.pallas_call(
        functools.partial(_experts_kernel, bt=BT, nt=NT),
        grid=grid,
        in_specs=[
            pl.BlockSpec((BT, H), hs_idx),
            pl.BlockSpec((1, H, CW), lambda e, c, t: (e, 0, c)),
            pl.BlockSpec((1, IW, H), lambda e, c, t: (e, c, 0)),
            pl.BlockSpec((1, 1, 1, CW), lambda e, c, t: (e, c, 0, 0)),
            pl.BlockSpec((E, H), lambda e, c, t: (0, 0)),
            pl.BlockSpec((1, BT, 1), lambda e, c, t: (e, t, 0)),
            pl.BlockSpec((BT, E), lambda e, c, t: (t, 0)),
        ],
        out_specs=pl.BlockSpec((T, H), lambda e, c, t: (0, 0)),
        out_shape=jax.ShapeDtypeStruct((T, H), jnp.float32),
        scratch_shapes=[
            pltpu.VMEM((T, HIDDEN), jnp.bfloat16),
            pltpu.VMEM((HIDDEN, CW), jnp.bfloat16),
            pltpu.VMEM((IW, HIDDEN), jnp.bfloat16),
            pltpu.VMEM((CW, CW), jnp.bfloat16),
        ],
        compiler_params=pltpu.CompilerParams(
            dimension_semantics=("arbitrary", "arbitrary", "arbitrary"),
            vmem_limit_bytes=64 * 1024 * 1024,
        ),
    )(hidden_states, gate_up_proj, down_proj, bgu, down_proj_bias, rw,
      routing_weights)
    return out
